# baseline (speedup 1.0000x reference)
import functools

import jax
import jax.numpy as jnp
from jax import lax
from jax.experimental import pallas as pl
from jax.experimental.pallas import tpu as pltpu, tpu_sc as plsc

DIM_VOCAB = 1000000
DIM_HIDDEN = 64
BATCH = 4096
HIST_LEN = 200

NUM_CORES = 2
NUM_SUBCORES = 16
NUM_WORKERS = NUM_CORES * NUM_SUBCORES

BPW = BATCH // NUM_WORKERS
IDX_PER_W = BPW * HIST_LEN
LANES = 16


def _gather_body(xf_hbm, tab_hbm, out_hbm,
                 xbuf, col0, col1, g0, g1, tb0, tb1,
                 gs0, gs1, os0, os1):
    cols = (col0, col1)
    gbufs = (g0, g1)
    tbufs = (tb0, tb1)
    gsem = (gs0, gs1)
    osem = (os0, os1)
    wid = lax.axis_index("s") * NUM_CORES + lax.axis_index("c")
    iota = lax.iota(jnp.int32, LANES)

    pltpu.sync_copy(xf_hbm.at[pl.ds(wid * IDX_PER_W, IDX_PER_W)], xbuf)

    def build_col(col, t):
        for g in range(BPW // LANES):
            v = plsc.load_gather(xbuf, [iota * HIST_LEN + (16 * g * HIST_LEN) + t])
            col[pl.ds(16 * g, LANES)] = v

    def transpose(gb, tb):
        def c_step(c, carry):
            for g in range(BPW // LANES):
                v = plsc.load_gather(gb, [iota + 16 * g, iota * 0 + c])
                tb[pl.ds(c * BPW + 16 * g, LANES)] = v
            return carry
        lax.fori_loop(0, DIM_HIDDEN, c_step, 0)

    def start_writes(tb, t, sem):
        for cg in range(DIM_HIDDEN // 8):
            pltpu.async_copy(tb.at[pl.ds(cg * 1024, 1024)],
                             out_hbm.at[t, cg, wid], sem)

    def drain_writes(tb, t, sem):
        for cg in range(DIM_HIDDEN // 8):
            pltpu.make_async_copy(tb.at[pl.ds(cg * 1024, 1024)],
                                  out_hbm.at[t, cg, wid], sem).wait()

    build_col(cols[0], 0)
    pltpu.async_copy(tab_hbm.at[cols[0]], gbufs[0], gsem[0])

    def step(i, carry):
        for b in range(2):
            t = 2 * i + b
            nb = 1 - b
            pltpu.make_async_copy(tab_hbm.at[cols[b]], gbufs[b],
                                  gsem[b]).wait()
            @pl.when(t + 1 < HIST_LEN)
            def _():
                build_col(cols[nb], t + 1)
                pltpu.async_copy(tab_hbm.at[cols[nb]], gbufs[nb], gsem[nb])
            @pl.when(t >= 2)
            def _():
                drain_writes(tbufs[b], t, osem[b])
            transpose(gbufs[b], tbufs[b])
            start_writes(tbufs[b], t, osem[b])
        return carry

    lax.fori_loop(0, HIST_LEN // 2, step, 0)

    for b in range(2):
        drain_writes(tbufs[b], HIST_LEN - 2 + b, osem[b])


@jax.jit
def _embed(x_flat, emb_lin):
    mesh = plsc.VectorSubcoreMesh(
        core_axis_name="c", subcore_axis_name="s",
        num_cores=NUM_CORES, num_subcores=NUM_SUBCORES)
    run = functools.partial(
        pl.kernel,
        mesh=mesh,
        compiler_params=pltpu.CompilerParams(
            use_tc_tiling_on_sc=False, needs_layout_passes=False),
        out_type=jax.ShapeDtypeStruct(
            (HIST_LEN, DIM_HIDDEN // 8, NUM_WORKERS, 8 * BPW), jnp.float32),
        scratch_types=[
            pltpu.VMEM((IDX_PER_W,), jnp.int32),
            pltpu.VMEM((BPW,), jnp.int32),
            pltpu.VMEM((BPW,), jnp.int32),
            pltpu.VMEM((BPW, DIM_HIDDEN), jnp.float32),
            pltpu.VMEM((BPW, DIM_HIDDEN), jnp.float32),
            pltpu.VMEM((BPW * DIM_HIDDEN,), jnp.float32),
            pltpu.VMEM((BPW * DIM_HIDDEN,), jnp.float32),
            pltpu.SemaphoreType.DMA,
            pltpu.SemaphoreType.DMA,
            pltpu.SemaphoreType.DMA,
            pltpu.SemaphoreType.DMA,
        ],
    )(_gather_body)
    return run(x_flat, emb_lin)


def kernel(x, emb_t):
    x_flat = x.reshape(-1).astype(jnp.int32)
    out4 = _embed(x_flat, emb_t)
    y = (out4.reshape(HIST_LEN, 8, NUM_WORKERS, 8, BPW)
         .transpose(2, 4, 0, 1, 3)
         .reshape(BATCH, HIST_LEN, DIM_HIDDEN))
    return y


# speedup vs baseline: 1.7809x; 1.7809x over previous
import functools

import jax
import jax.numpy as jnp
from jax import lax
from jax.experimental import pallas as pl
from jax.experimental.pallas import tpu as pltpu, tpu_sc as plsc

DIM_VOCAB = 1000000
DIM_HIDDEN = 64
BATCH = 4096
HIST_LEN = 200

NUM_CORES = 2
NUM_SUBCORES = 16
NUM_WORKERS = NUM_CORES * NUM_SUBCORES

BPW = BATCH // NUM_WORKERS
IDX_PER_W = BPW * HIST_LEN
LANES = 16


def _gather_body(xf_hbm, tab_hbm, out_hbm,
                 xbuf, col0, col1, g0, g1, tb0, tb1,
                 gs0, gs1, os0, os1):
    cols = (col0, col1)
    gbufs = (g0, g1)
    tbufs = (tb0, tb1)
    gsem = (gs0, gs1)
    osem = (os0, os1)
    wid = lax.axis_index("s") * NUM_CORES + lax.axis_index("c")
    iota = lax.iota(jnp.int32, LANES)

    pltpu.sync_copy(xf_hbm.at[pl.ds(wid * IDX_PER_W, IDX_PER_W)], xbuf)

    def build_col(col, t):
        for g in range(BPW // LANES):
            v = plsc.load_gather(xbuf, [iota * HIST_LEN + (16 * g * HIST_LEN) + t])
            col[pl.ds(16 * g, LANES)] = v

    def transpose(gb, tb):
        def blk(i, carry):
            j0 = (i % 8) * LANES
            c0 = (i // 8) * LANES
            base_w = c0 * BPW + j0
            for k in range(LANES):
                rem = (iota + k) & (LANES - 1)
                v = plsc.load_gather(gb, [iota + j0, rem + c0])
                plsc.store_scatter(tb, [rem * BPW + iota + base_w], v)
            return carry
        lax.fori_loop(0, (BPW // LANES) * (DIM_HIDDEN // LANES), blk, 0)

    def start_writes(tb, t, sem):
        for cg in range(DIM_HIDDEN // 8):
            pltpu.async_copy(tb.at[pl.ds(cg * 1024, 1024)],
                             out_hbm.at[t, cg, wid], sem)

    def drain_writes(tb, t, sem):
        for cg in range(DIM_HIDDEN // 8):
            pltpu.make_async_copy(tb.at[pl.ds(cg * 1024, 1024)],
                                  out_hbm.at[t, cg, wid], sem).wait()

    build_col(cols[0], 0)
    pltpu.async_copy(tab_hbm.at[cols[0]], gbufs[0], gsem[0])

    def step(i, carry):
        for b in range(2):
            t = 2 * i + b
            nb = 1 - b
            pltpu.make_async_copy(tab_hbm.at[cols[b]], gbufs[b],
                                  gsem[b]).wait()
            @pl.when(t + 1 < HIST_LEN)
            def _():
                build_col(cols[nb], t + 1)
                pltpu.async_copy(tab_hbm.at[cols[nb]], gbufs[nb], gsem[nb])
            @pl.when(t >= 2)
            def _():
                drain_writes(tbufs[b], t, osem[b])
            transpose(gbufs[b], tbufs[b])
            start_writes(tbufs[b], t, osem[b])
        return carry

    lax.fori_loop(0, HIST_LEN // 2, step, 0)

    for b in range(2):
        drain_writes(tbufs[b], HIST_LEN - 2 + b, osem[b])


@jax.jit
def _embed(x_flat, emb_lin):
    mesh = plsc.VectorSubcoreMesh(
        core_axis_name="c", subcore_axis_name="s",
        num_cores=NUM_CORES, num_subcores=NUM_SUBCORES)
    run = functools.partial(
        pl.kernel,
        mesh=mesh,
        compiler_params=pltpu.CompilerParams(
            use_tc_tiling_on_sc=False, needs_layout_passes=False),
        out_type=jax.ShapeDtypeStruct(
            (HIST_LEN, DIM_HIDDEN // 8, NUM_WORKERS, 8 * BPW), jnp.float32),
        scratch_types=[
            pltpu.VMEM((IDX_PER_W,), jnp.int32),
            pltpu.VMEM((BPW,), jnp.int32),
            pltpu.VMEM((BPW,), jnp.int32),
            pltpu.VMEM((BPW, DIM_HIDDEN), jnp.float32),
            pltpu.VMEM((BPW, DIM_HIDDEN), jnp.float32),
            pltpu.VMEM((BPW * DIM_HIDDEN,), jnp.float32),
            pltpu.VMEM((BPW * DIM_HIDDEN,), jnp.float32),
            pltpu.SemaphoreType.DMA,
            pltpu.SemaphoreType.DMA,
            pltpu.SemaphoreType.DMA,
            pltpu.SemaphoreType.DMA,
        ],
    )(_gather_body)
    return run(x_flat, emb_lin)


def kernel(x, emb_t):
    x_flat = x.reshape(-1).astype(jnp.int32)
    out4 = _embed(x_flat, emb_t)
    y = (out4.reshape(HIST_LEN, 8, NUM_WORKERS, 8, BPW)
         .transpose(2, 4, 0, 1, 3)
         .reshape(BATCH, HIST_LEN, DIM_HIDDEN))
    return y


# speedup vs baseline: 1.9564x; 1.0986x over previous
import functools

import jax
import jax.numpy as jnp
from jax import lax
from jax.experimental import pallas as pl
from jax.experimental.pallas import tpu as pltpu, tpu_sc as plsc

DIM_VOCAB = 1000000
DIM_HIDDEN = 64
BATCH = 4096
HIST_LEN = 200

NUM_CORES = 2
NUM_SUBCORES = 16
NUM_WORKERS = NUM_CORES * NUM_SUBCORES

BPW = BATCH // NUM_WORKERS
IDX_PER_W = BPW * HIST_LEN
LANES = 16

VCOLS = 7813
VOCAB_PAD = VCOLS * 128
COLS_PER_W = 245


def _detile_body(tabT_hbm, lin_hbm, b0, b1, o0, o1, rs0, rs1, ws0, ws1):
    bufs = (b0, b1)
    obufs = (o0, o1)
    rsem = (rs0, rs1)
    wsem = (ws0, ws1)
    wid = lax.axis_index("s") * NUM_CORES + lax.axis_index("c")
    iota = lax.iota(jnp.int32, LANES)

    def read_start(b, vc):
        pltpu.async_copy(tabT_hbm.at[:, pl.ds(vc * 128, 128)], bufs[b],
                         rsem[b])

    def transpose_block(bf, ob):
        def blk(i, carry):
            c0 = (i % 4) * LANES
            v0 = (i // 4) * LANES
            for k in range(LANES):
                rem = (iota + k) & (LANES - 1)
                v = plsc.load_gather(bf, [iota + c0, rem + v0])
                plsc.store_scatter(ob, [(rem + v0) * DIM_HIDDEN + c0 + iota],
                                   v)
            return carry
        lax.fori_loop(0, (DIM_HIDDEN // LANES) * (128 // LANES), blk, 0)

    def write_start(b, vc):
        pltpu.async_copy(obufs[b], lin_hbm.at[pl.ds(vc * 8192, 8192)],
                         wsem[b])

    def drain_write(b):
        pltpu.make_async_copy(obufs[b], lin_hbm.at[pl.ds(0, 8192)],
                              wsem[b]).wait()

    for b in range(2):
        read_start(b, wid + 32 * b)

    def step(g, carry):
        for b in range(2):
            i = 2 * g + b
            vc = wid + 32 * i

            @pl.when((i < COLS_PER_W) & (vc < VCOLS))
            def _():
                pltpu.make_async_copy(
                    tabT_hbm.at[:, pl.ds(vc * 128, 128)], bufs[b],
                    rsem[b]).wait()
                @pl.when(i >= 2)
                def _():
                    drain_write(b)
                transpose_block(bufs[b], obufs[b])
                write_start(b, vc)
                nvc = vc + 64
                @pl.when((i + 2 < COLS_PER_W) & (nvc < VCOLS))
                def _():
                    read_start(b, nvc)
        return carry

    lax.fori_loop(0, (COLS_PER_W + 1) // 2, step, 0)

    for b in range(2):
        drain_write(b)


def _gather_body(xf_hbm, tab_hbm, out_hbm,
                 xbuf, col0, col1, g0, g1, tb0, tb1,
                 gs0, gs1, os0, os1):
    cols = (col0, col1)
    gbufs = (g0, g1)
    tbufs = (tb0, tb1)
    gsem = (gs0, gs1)
    osem = (os0, os1)
    wid = lax.axis_index("s") * NUM_CORES + lax.axis_index("c")
    iota = lax.iota(jnp.int32, LANES)

    pltpu.sync_copy(xf_hbm.at[pl.ds(wid * IDX_PER_W, IDX_PER_W)], xbuf)

    def build_col(col, t):
        for g in range(BPW // LANES):
            v = plsc.load_gather(xbuf, [iota * HIST_LEN + (16 * g * HIST_LEN) + t])
            col[pl.ds(16 * g, LANES)] = v

    def transpose(gb, tb):
        def blk(i, carry):
            j0 = (i % 8) * LANES
            c0 = (i // 8) * LANES
            base_w = c0 * BPW + j0
            for k in range(LANES):
                rem = (iota + k) & (LANES - 1)
                v = plsc.load_gather(gb, [iota + j0, rem + c0])
                plsc.store_scatter(tb, [rem * BPW + iota + base_w], v)
            return carry
        lax.fori_loop(0, (BPW // LANES) * (DIM_HIDDEN // LANES), blk, 0)

    def start_writes(tb, t, sem):
        for cg in range(DIM_HIDDEN // 8):
            pltpu.async_copy(tb.at[pl.ds(cg * 1024, 1024)],
                             out_hbm.at[t, cg, wid], sem)

    def drain_writes(tb, t, sem):
        for cg in range(DIM_HIDDEN // 8):
            pltpu.make_async_copy(tb.at[pl.ds(cg * 1024, 1024)],
                                  out_hbm.at[t, cg, wid], sem).wait()

    build_col(cols[0], 0)
    pltpu.async_copy(tab_hbm.at[cols[0]], gbufs[0], gsem[0])

    def step(i, carry):
        for b in range(2):
            t = 2 * i + b
            nb = 1 - b
            pltpu.make_async_copy(tab_hbm.at[cols[b]], gbufs[b],
                                  gsem[b]).wait()
            @pl.when(t + 1 < HIST_LEN)
            def _():
                build_col(cols[nb], t + 1)
                pltpu.async_copy(tab_hbm.at[cols[nb]], gbufs[nb], gsem[nb])
            @pl.when(t >= 2)
            def _():
                drain_writes(tbufs[b], t, osem[b])
            transpose(gbufs[b], tbufs[b])
            start_writes(tbufs[b], t, osem[b])
        return carry

    lax.fori_loop(0, HIST_LEN // 2, step, 0)

    for b in range(2):
        drain_writes(tbufs[b], HIST_LEN - 2 + b, osem[b])


@jax.jit
def _embed(x_flat, emb_T):
    mesh = plsc.VectorSubcoreMesh(
        core_axis_name="c", subcore_axis_name="s",
        num_cores=NUM_CORES, num_subcores=NUM_SUBCORES)
    detile = functools.partial(
        pl.kernel,
        mesh=mesh,
        compiler_params=pltpu.CompilerParams(
            use_tc_tiling_on_sc=True, needs_layout_passes=False),
        out_type=jax.ShapeDtypeStruct((VOCAB_PAD * DIM_HIDDEN,), jnp.float32),
        scratch_types=[
            pltpu.VMEM((DIM_HIDDEN, 128), jnp.float32),
            pltpu.VMEM((DIM_HIDDEN, 128), jnp.float32),
            pltpu.VMEM((128 * DIM_HIDDEN,), jnp.float32),
            pltpu.VMEM((128 * DIM_HIDDEN,), jnp.float32),
            pltpu.SemaphoreType.DMA,
            pltpu.SemaphoreType.DMA,
            pltpu.SemaphoreType.DMA,
            pltpu.SemaphoreType.DMA,
        ],
    )(_detile_body)
    emb_lin = detile(emb_T).reshape(VOCAB_PAD, DIM_HIDDEN)
    run = functools.partial(
        pl.kernel,
        mesh=mesh,
        compiler_params=pltpu.CompilerParams(
            use_tc_tiling_on_sc=False, needs_layout_passes=False),
        out_type=jax.ShapeDtypeStruct(
            (HIST_LEN, DIM_HIDDEN // 8, NUM_WORKERS, 8 * BPW), jnp.float32),
        scratch_types=[
            pltpu.VMEM((IDX_PER_W,), jnp.int32),
            pltpu.VMEM((BPW,), jnp.int32),
            pltpu.VMEM((BPW,), jnp.int32),
            pltpu.VMEM((BPW, DIM_HIDDEN), jnp.float32),
            pltpu.VMEM((BPW, DIM_HIDDEN), jnp.float32),
            pltpu.VMEM((BPW * DIM_HIDDEN,), jnp.float32),
            pltpu.VMEM((BPW * DIM_HIDDEN,), jnp.float32),
            pltpu.SemaphoreType.DMA,
            pltpu.SemaphoreType.DMA,
            pltpu.SemaphoreType.DMA,
            pltpu.SemaphoreType.DMA,
        ],
    )(_gather_body)
    return run(x_flat, emb_lin)


def kernel(x, emb_t):
    x_flat = x.reshape(-1).astype(jnp.int32)
    out4 = _embed(x_flat, emb_t.T)
    y = (out4.reshape(HIST_LEN, 8, NUM_WORKERS, 8, BPW)
         .transpose(2, 4, 0, 1, 3)
         .reshape(BATCH, HIST_LEN, DIM_HIDDEN))
    return y


# speedup vs baseline: 1.9645x; 1.0041x over previous
import functools

import jax
import jax.numpy as jnp
from jax import lax
from jax.experimental import pallas as pl
from jax.experimental.pallas import tpu as pltpu, tpu_sc as plsc

DIM_VOCAB = 1000000
DIM_HIDDEN = 64
BATCH = 4096
HIST_LEN = 200

NUM_CORES = 2
NUM_SUBCORES = 16
NUM_WORKERS = NUM_CORES * NUM_SUBCORES

BPW = BATCH // NUM_WORKERS
IDX_PER_W = BPW * HIST_LEN
LANES = 16

VCOLS = 7813
VOCAB_PAD = VCOLS * 128
COLS_PER_W = 245


def _detile_body(tabT_hbm, lin_hbm, b0, b1, o0, o1, rs0, rs1, ws0, ws1):
    bufs = (b0, b1)
    obufs = (o0, o1)
    rsem = (rs0, rs1)
    wsem = (ws0, ws1)
    wid = lax.axis_index("s") * NUM_CORES + lax.axis_index("c")
    iota = lax.iota(jnp.int32, LANES)

    def read_start(b, vc):
        pltpu.async_copy(tabT_hbm.at[:, pl.ds(vc * 128, 128)], bufs[b],
                         rsem[b])

    def transpose_block(bf, ob):
        def blk(i, carry):
            c0 = (i % 4) * LANES
            v0 = (i // 4) * LANES
            for k in range(LANES):
                rem = (iota + k) & (LANES - 1)
                v = plsc.load_gather(bf, [iota + c0, rem + v0])
                plsc.store_scatter(ob, [(rem + v0) * DIM_HIDDEN + c0 + iota],
                                   v)
            return carry
        lax.fori_loop(0, (DIM_HIDDEN // LANES) * (128 // LANES), blk, 0)

    def write_start(b, vc):
        pltpu.async_copy(obufs[b], lin_hbm.at[pl.ds(vc * 8192, 8192)],
                         wsem[b])

    def drain_write(b):
        pltpu.make_async_copy(obufs[b], lin_hbm.at[pl.ds(0, 8192)],
                              wsem[b]).wait()

    for b in range(2):
        read_start(b, wid + 32 * b)

    def step(g, carry):
        for b in range(2):
            i = 2 * g + b
            vc = wid + 32 * i

            @pl.when((i < COLS_PER_W) & (vc < VCOLS))
            def _():
                pltpu.make_async_copy(
                    tabT_hbm.at[:, pl.ds(vc * 128, 128)], bufs[b],
                    rsem[b]).wait()
                @pl.when(i >= 2)
                def _():
                    drain_write(b)
                transpose_block(bufs[b], obufs[b])
                write_start(b, vc)
                nvc = vc + 64
                @pl.when((i + 2 < COLS_PER_W) & (nvc < VCOLS))
                def _():
                    read_start(b, nvc)
        return carry

    lax.fori_loop(0, (COLS_PER_W + 1) // 2, step, 0)

    for b in range(2):
        drain_write(b)


def _gather_body(xf_hbm, tab_hbm, out_hbm,
                 xbuf, col0, col1, g0, g1, tb0, tb1,
                 gs0, gs1, os0, os1):
    cols = (col0, col1)
    gbufs = (g0, g1)
    tbufs = (tb0, tb1)
    gsem = (gs0, gs1)
    osem = (os0, os1)
    wid = lax.axis_index("s") * NUM_CORES + lax.axis_index("c")
    iota = lax.iota(jnp.int32, LANES)

    pltpu.sync_copy(xf_hbm.at[pl.ds(wid * IDX_PER_W, IDX_PER_W)], xbuf)

    def build_col(col, t):
        for g in range(BPW // LANES):
            v = plsc.load_gather(xbuf, [iota * HIST_LEN + (16 * g * HIST_LEN) + t])
            col[pl.ds(16 * g, LANES)] = v

    def transpose(gb, tb):
        def j_step(j, carry):
            for g in range(DIM_HIDDEN // LANES):
                v = gb[j, pl.ds(g * LANES, LANES)]
                plsc.store_scatter(tb, [iota + g * LANES, (iota & 0) + j], v)
            return carry
        lax.fori_loop(0, BPW, j_step, 0)

    def start_writes(tb, t, sem):
        for cg in range(DIM_HIDDEN // 8):
            pltpu.async_copy(tb.at[pl.ds(cg * 8, 8), pl.ds(0, BPW)],
                             out_hbm.at[t, cg, wid], sem)

    def drain_writes(tb, t, sem):
        for cg in range(DIM_HIDDEN // 8):
            pltpu.make_async_copy(tb.at[pl.ds(cg * 8, 8), pl.ds(0, BPW)],
                                  out_hbm.at[t, cg, wid], sem).wait()

    build_col(cols[0], 0)
    pltpu.async_copy(tab_hbm.at[cols[0]], gbufs[0], gsem[0])

    def step(i, carry):
        for b in range(2):
            t = 2 * i + b
            nb = 1 - b
            pltpu.make_async_copy(tab_hbm.at[cols[b]], gbufs[b],
                                  gsem[b]).wait()
            @pl.when(t + 1 < HIST_LEN)
            def _():
                build_col(cols[nb], t + 1)
                pltpu.async_copy(tab_hbm.at[cols[nb]], gbufs[nb], gsem[nb])
            @pl.when(t >= 2)
            def _():
                drain_writes(tbufs[b], t, osem[b])
            transpose(gbufs[b], tbufs[b])
            start_writes(tbufs[b], t, osem[b])
        return carry

    lax.fori_loop(0, HIST_LEN // 2, step, 0)

    for b in range(2):
        drain_writes(tbufs[b], HIST_LEN - 2 + b, osem[b])


@jax.jit
def _embed(x_flat, emb_T):
    mesh = plsc.VectorSubcoreMesh(
        core_axis_name="c", subcore_axis_name="s",
        num_cores=NUM_CORES, num_subcores=NUM_SUBCORES)
    detile = functools.partial(
        pl.kernel,
        mesh=mesh,
        compiler_params=pltpu.CompilerParams(
            use_tc_tiling_on_sc=True, needs_layout_passes=False),
        out_type=jax.ShapeDtypeStruct((VOCAB_PAD * DIM_HIDDEN,), jnp.float32),
        scratch_types=[
            pltpu.VMEM((DIM_HIDDEN, 128), jnp.float32),
            pltpu.VMEM((DIM_HIDDEN, 128), jnp.float32),
            pltpu.VMEM((128 * DIM_HIDDEN,), jnp.float32),
            pltpu.VMEM((128 * DIM_HIDDEN,), jnp.float32),
            pltpu.SemaphoreType.DMA,
            pltpu.SemaphoreType.DMA,
            pltpu.SemaphoreType.DMA,
            pltpu.SemaphoreType.DMA,
        ],
    )(_detile_body)
    emb_lin = detile(emb_T).reshape(VOCAB_PAD, DIM_HIDDEN)
    run = functools.partial(
        pl.kernel,
        mesh=mesh,
        compiler_params=pltpu.CompilerParams(
            use_tc_tiling_on_sc=False, needs_layout_passes=False),
        out_type=jax.ShapeDtypeStruct(
            (HIST_LEN, DIM_HIDDEN // 8, NUM_WORKERS, 8, BPW), jnp.float32),
        scratch_types=[
            pltpu.VMEM((IDX_PER_W,), jnp.int32),
            pltpu.VMEM((BPW,), jnp.int32),
            pltpu.VMEM((BPW,), jnp.int32),
            pltpu.VMEM((BPW, DIM_HIDDEN), jnp.float32),
            pltpu.VMEM((BPW, DIM_HIDDEN), jnp.float32),
            pltpu.VMEM((DIM_HIDDEN, BPW + 1), jnp.float32),
            pltpu.VMEM((DIM_HIDDEN, BPW + 1), jnp.float32),
            pltpu.SemaphoreType.DMA,
            pltpu.SemaphoreType.DMA,
            pltpu.SemaphoreType.DMA,
            pltpu.SemaphoreType.DMA,
        ],
    )(_gather_body)
    return run(x_flat, emb_lin)


def kernel(x, emb_t):
    x_flat = x.reshape(-1).astype(jnp.int32)
    out5 = _embed(x_flat, emb_t.T)
    y = (out5.transpose(2, 4, 0, 1, 3)
         .reshape(BATCH, HIST_LEN, DIM_HIDDEN))
    return y
